# Initial kernel scaffold; baseline (speedup 1.0000x reference)
import functools

import jax
import jax.numpy as jnp
from jax import lax
from jax.experimental import pallas as pl
from jax.experimental.pallas import tpu as pltpu
from jax.experimental.pallas import tpu_sc as plsc

N = 10000
D = 128
E = 320000

NC = 2
NS = 16
NW = NC * NS
L = 16

CHUNK = 128
CHUNKS = 80
EPT = CHUNKS * CHUNK
E_PAD = NW * EPT
N1 = 10240
RPT = N1 // NS

_mesh = plsc.VectorSubcoreMesh(
    core_axis_name="c", subcore_axis_name="s", num_cores=NC, num_subcores=NS
)


@functools.partial(
    pl.kernel,
    out_type=jax.ShapeDtypeStruct((NC, N1), jnp.float32),
    mesh=_mesh,
    scratch_types=[
        pltpu.VMEM((CHUNKS, CHUNK), jnp.int32),
        pltpu.VMEM((CHUNK,), jnp.float32),
        pltpu.VMEM((RPT,), jnp.float32),
        pltpu.VMEM_SHARED((N1,), jnp.float32),
    ],
)
def _deg_kernel(dst_hbm, out_hbm, idx_v, ones_v, zero_v, deg_sh):
    c = lax.axis_index("c")
    s = lax.axis_index("s")
    w = c * NS + s

    for i in range(CHUNK // L):
        ones_v[pl.ds(i * L, L)] = jnp.ones((L,), jnp.float32)
    for i in range(RPT // L):
        zero_v[pl.ds(i * L, L)] = jnp.zeros((L,), jnp.float32)

    pltpu.sync_copy(dst_hbm.at[w], idx_v)
    pltpu.sync_copy(zero_v, deg_sh.at[pl.ds(s * RPT, RPT)])
    plsc.subcore_barrier()

    def body(j, carry):
        pltpu.sync_copy(ones_v, deg_sh.at[idx_v.at[j]], add=True)
        return carry

    lax.fori_loop(0, CHUNKS, body, 0)
    plsc.subcore_barrier()
    pltpu.sync_copy(deg_sh.at[pl.ds(s * RPT, RPT)],
                    out_hbm.at[c, pl.ds(s * RPT, RPT)])


@functools.partial(
    pl.kernel,
    out_type=jax.ShapeDtypeStruct((NC, N1, D), jnp.float32),
    mesh=_mesh,
    scratch_types=[
        pltpu.VMEM((CHUNKS, CHUNK), jnp.int32),
        pltpu.VMEM((CHUNKS, CHUNK), jnp.int32),
        pltpu.VMEM((2, CHUNK, D), jnp.float32),
        pltpu.VMEM_SHARED((N1, D), jnp.float32),
        pltpu.SemaphoreType.DMA,
        pltpu.SemaphoreType.DMA,
    ],
)
def _agg_kernel(y_hbm, src_hbm, dst_hbm, out_hbm, src_v, dst_v, buf_v,
                acc_sh, gsem0, gsem1):
    c = lax.axis_index("c")
    s = lax.axis_index("s")
    w = c * NS + s
    base = s * RPT

    pltpu.sync_copy(src_hbm.at[w], src_v)
    pltpu.sync_copy(dst_hbm.at[w], dst_v)
    pltpu.sync_copy(y_hbm.at[pl.ds(base, RPT)], acc_sh.at[pl.ds(base, RPT)])
    plsc.subcore_barrier()

    def gather(j, b, sem):
        return pltpu.make_async_copy(y_hbm.at[src_v.at[j]], buf_v.at[b], sem)

    gather(0, 0, gsem0).start()

    def pair(jj, carry):
        j0 = 2 * jj
        j1 = j0 + 1
        gather(j0, 0, gsem0).wait()
        gather(j1, 1, gsem1).start()
        pltpu.sync_copy(buf_v.at[0], acc_sh.at[dst_v.at[j0]], add=True)
        gather(j1, 1, gsem1).wait()

        @pl.when(jj < CHUNKS // 2 - 1)
        def _():
            gather(j0 + 2, 0, gsem0).start()

        pltpu.sync_copy(buf_v.at[1], acc_sh.at[dst_v.at[j1]], add=True)
        return carry

    lax.fori_loop(0, CHUNKS // 2, pair, 0)
    plsc.subcore_barrier()
    pltpu.sync_copy(acc_sh.at[pl.ds(base, RPT)],
                    out_hbm.at[c, pl.ds(base, RPT)])


def _prescale_body(x_ref, w_ref, dp_ref, y_ref):
    xw = jnp.dot(x_ref[...], w_ref[...], preferred_element_type=jnp.float32)
    deg = dp_ref[0] + dp_ref[1] + 1.0
    y_ref[...] = xw * lax.rsqrt(deg)


def _epilogue_body(a_ref, y_ref, dp_ref, x_ref, b_ref, bw_ref, bb_ref,
                   bm_ref, bv_ref, o_ref):
    deg = dp_ref[0] + dp_ref[1] + 1.0
    dinv = lax.rsqrt(deg)
    agg = (a_ref[0] + a_ref[1] - y_ref[...]) * dinv + b_ref[...]
    inv_std = lax.rsqrt(bv_ref[...] + 1e-5)
    h = (agg - bm_ref[...]) * inv_std * bw_ref[...] + bb_ref[...]
    o_ref[...] = jnp.maximum(h, 0.0) + x_ref[...]


BR = 1024
_GRID = N1 // BR


def kernel(x, edge_index, W, b, bn_weight, bn_bias, bn_mean, bn_var):
    pad_e = E_PAD - E
    src = jnp.concatenate(
        [edge_index[0], jnp.full((pad_e,), N, jnp.int32)]).reshape(
            NW, CHUNKS, CHUNK)
    dst = jnp.concatenate(
        [edge_index[1], jnp.full((pad_e,), N, jnp.int32)]).reshape(
            NW, CHUNKS, CHUNK)
    x_pad = jnp.pad(x, ((0, N1 - N), (0, 0)))

    deg_parts = _deg_kernel(dst)
    dp3 = deg_parts.reshape(NC, N1, 1)

    y = pl.pallas_call(
        _prescale_body,
        grid=(_GRID,),
        in_specs=[
            pl.BlockSpec((BR, D), lambda i: (i, 0)),
            pl.BlockSpec((D, D), lambda i: (0, 0)),
            pl.BlockSpec((NC, BR, 1), lambda i: (0, i, 0)),
        ],
        out_specs=pl.BlockSpec((BR, D), lambda i: (i, 0)),
        out_shape=jax.ShapeDtypeStruct((N1, D), jnp.float32),
    )(x_pad, W, dp3)

    agg_parts = _agg_kernel(y, src, dst)

    vec = lambda a: a.reshape(1, D)
    h = pl.pallas_call(
        _epilogue_body,
        grid=(_GRID,),
        in_specs=[
            pl.BlockSpec((NC, BR, D), lambda i: (0, i, 0)),
            pl.BlockSpec((BR, D), lambda i: (i, 0)),
            pl.BlockSpec((NC, BR, 1), lambda i: (0, i, 0)),
            pl.BlockSpec((BR, D), lambda i: (i, 0)),
        ] + [pl.BlockSpec((1, D), lambda i: (0, 0))] * 5,
        out_specs=pl.BlockSpec((BR, D), lambda i: (i, 0)),
        out_shape=jax.ShapeDtypeStruct((N1, D), jnp.float32),
    )(agg_parts, y, dp3, x_pad, vec(b), vec(bn_weight), vec(bn_bias),
      vec(bn_mean), vec(bn_var))

    return h[:N]


# speedup vs baseline: 12.4240x; 12.4240x over previous
import functools

import jax
import jax.numpy as jnp
from jax import lax
from jax.experimental import pallas as pl
from jax.experimental.pallas import tpu as pltpu
from jax.experimental.pallas import tpu_sc as plsc

N = 10000
D = 128
E = 320000

NC = 2
NS = 16
NW = NC * NS
L = 16

CHUNK = 128
CHUNKS = 80
BLK = 16
E_PAD = NW * CHUNKS * CHUNK
N1 = 10240
RPT = N1 // NS

_mesh = plsc.VectorSubcoreMesh(
    core_axis_name="c", subcore_axis_name="s", num_cores=NC, num_subcores=NS
)


@functools.partial(
    pl.kernel,
    out_type=jax.ShapeDtypeStruct((NC, N1), jnp.float32),
    mesh=_mesh,
    scratch_types=[
        pltpu.VMEM((CHUNKS, CHUNK), jnp.int32),
        pltpu.VMEM((CHUNK,), jnp.float32),
        pltpu.VMEM((RPT,), jnp.float32),
        pltpu.VMEM_SHARED((N1,), jnp.float32),
    ],
)
def _deg_kernel(dst_hbm, out_hbm, idx_v, ones_v, zero_v, deg_sh):
    c = lax.axis_index("c")
    s = lax.axis_index("s")
    w = c * NS + s

    for i in range(CHUNK // L):
        ones_v[pl.ds(i * L, L)] = jnp.ones((L,), jnp.float32)
    for i in range(RPT // L):
        zero_v[pl.ds(i * L, L)] = jnp.zeros((L,), jnp.float32)

    pltpu.sync_copy(dst_hbm.at[w], idx_v)
    pltpu.sync_copy(zero_v, deg_sh.at[pl.ds(s * RPT, RPT)])
    plsc.subcore_barrier()

    def body(j, carry):
        pltpu.sync_copy(ones_v, deg_sh.at[idx_v.at[j]], add=True)
        return carry

    lax.fori_loop(0, CHUNKS, body, 0)
    plsc.subcore_barrier()
    pltpu.sync_copy(deg_sh.at[pl.ds(s * RPT, RPT)],
                    out_hbm.at[c, pl.ds(s * RPT, RPT)])


@functools.partial(
    pl.kernel,
    out_type=jax.ShapeDtypeStruct((NC, N1, D), jnp.float32),
    mesh=_mesh,
    scratch_types=[
        pltpu.VMEM((CHUNKS, CHUNK), jnp.int32),
        pltpu.VMEM((BLK, CHUNK), jnp.int32),
        pltpu.VMEM((2, CHUNK, D), jnp.float32),
        pltpu.VMEM_SHARED((N1, D), jnp.float32),
        pltpu.SemaphoreType.DMA,
        pltpu.SemaphoreType.DMA,
    ],
)
def _agg_kernel(y_hbm, src_hbm, dst_hbm, out_hbm, src_v, dst_v, buf_v,
                acc_sh, gsem0, gsem1):
    c = lax.axis_index("c")
    s = lax.axis_index("s")
    w = c * NS + s
    base = s * RPT

    pltpu.sync_copy(src_hbm.at[w], src_v)
    pltpu.sync_copy(y_hbm.at[pl.ds(base, RPT)], acc_sh.at[pl.ds(base, RPT)])
    plsc.subcore_barrier()

    def gather(j, b, sem):
        return pltpu.make_async_copy(y_hbm.at[src_v.at[j]], buf_v.at[b], sem)

    def scatter(jb, b):
        pltpu.sync_copy(buf_v.at[b], acc_sh.at[dst_v.at[jb]], add=True)

    gather(0, 0, gsem0).start()
    gather(1, 1, gsem1).start()

    last_blk = CHUNKS // BLK - 1
    for blk in range(CHUNKS // BLK):
        pltpu.sync_copy(dst_hbm.at[w, pl.ds(blk * BLK, BLK)], dst_v)

        def pair(jj, carry, blk=blk):
            j0 = blk * BLK + 2 * jj
            j1 = j0 + 1
            gather(j0, 0, gsem0).wait()
            scatter(2 * jj, 0)
            if blk < last_blk:
                gather(j0 + 2, 0, gsem0).start()
            else:
                @pl.when(jj < BLK // 2 - 1)
                def _():
                    gather(j0 + 2, 0, gsem0).start()
            gather(j1, 1, gsem1).wait()
            scatter(2 * jj + 1, 1)
            if blk < last_blk:
                gather(j1 + 2, 1, gsem1).start()
            else:
                @pl.when(jj < BLK // 2 - 1)
                def _():
                    gather(j1 + 2, 1, gsem1).start()
            return carry

        lax.fori_loop(0, BLK // 2, pair, 0)

    plsc.subcore_barrier()
    pltpu.sync_copy(acc_sh.at[pl.ds(base, RPT)],
                    out_hbm.at[c, pl.ds(base, RPT)])


def _prescale_body(x_ref, w_ref, dp_ref, y_ref):
    xw = jnp.dot(x_ref[...], w_ref[...], preferred_element_type=jnp.float32)
    deg = dp_ref[0] + dp_ref[1] + 1.0
    y_ref[...] = xw * lax.rsqrt(deg)


def _epilogue_body(a_ref, y_ref, dp_ref, x_ref, b_ref, bw_ref, bb_ref,
                   bm_ref, bv_ref, o_ref):
    deg = dp_ref[0] + dp_ref[1] + 1.0
    dinv = lax.rsqrt(deg)
    agg = (a_ref[0] + a_ref[1] - y_ref[...]) * dinv + b_ref[...]
    inv_std = lax.rsqrt(bv_ref[...] + 1e-5)
    h = (agg - bm_ref[...]) * inv_std * bw_ref[...] + bb_ref[...]
    o_ref[...] = jnp.maximum(h, 0.0) + x_ref[...]


BR = 1024
_GRID = N1 // BR


def kernel(x, edge_index, W, b, bn_weight, bn_bias, bn_mean, bn_var):
    pad_e = E_PAD - E
    src_flat = jnp.concatenate(
        [edge_index[0], jnp.full((pad_e,), N, jnp.int32)])
    dst_flat = jnp.concatenate(
        [edge_index[1], jnp.full((pad_e,), N, jnp.int32)])
    dst_t = dst_flat.reshape(NW, CHUNKS, CHUNK)
    src_t = src_flat.reshape(NW, CHUNKS, CHUNK)
    x_pad = jnp.pad(x, ((0, N1 - N), (0, 0)))

    deg_parts = _deg_kernel(dst_t)
    dp3 = deg_parts.reshape(NC, N1, 1)

    y = pl.pallas_call(
        _prescale_body,
        grid=(_GRID,),
        in_specs=[
            pl.BlockSpec((BR, D), lambda i: (i, 0)),
            pl.BlockSpec((D, D), lambda i: (0, 0)),
            pl.BlockSpec((NC, BR, 1), lambda i: (0, i, 0)),
        ],
        out_specs=pl.BlockSpec((BR, D), lambda i: (i, 0)),
        out_shape=jax.ShapeDtypeStruct((N1, D), jnp.float32),
    )(x_pad, W, dp3)

    agg_parts = _agg_kernel(y, src_t, dst_t)

    vec = lambda a: a.reshape(1, D)
    h = pl.pallas_call(
        _epilogue_body,
        grid=(_GRID,),
        in_specs=[
            pl.BlockSpec((NC, BR, D), lambda i: (0, i, 0)),
            pl.BlockSpec((BR, D), lambda i: (i, 0)),
            pl.BlockSpec((NC, BR, 1), lambda i: (0, i, 0)),
            pl.BlockSpec((BR, D), lambda i: (i, 0)),
        ] + [pl.BlockSpec((1, D), lambda i: (0, 0))] * 5,
        out_specs=pl.BlockSpec((BR, D), lambda i: (i, 0)),
        out_shape=jax.ShapeDtypeStruct((N1, D), jnp.float32),
    )(agg_parts, y, dp3, x_pad, vec(b), vec(bn_weight), vec(bn_bias),
      vec(bn_mean), vec(bn_var))

    return h[:N]


# speedup vs baseline: 36.2025x; 2.9139x over previous
import functools

import jax
import jax.numpy as jnp
from jax import lax
from jax.experimental import pallas as pl
from jax.experimental.pallas import tpu as pltpu
from jax.experimental.pallas import tpu_sc as plsc

N = 10000
D = 128
E = 320000

NC = 2
NS = 16
NW = NC * NS
L = 16

CHUNK = 128
CHUNKS = 80
BLK = 16
E_PAD = NW * CHUNKS * CHUNK
N1 = 10240
RPT = N1 // NS

_mesh = plsc.VectorSubcoreMesh(
    core_axis_name="c", subcore_axis_name="s", num_cores=NC, num_subcores=NS
)


@functools.partial(
    pl.kernel,
    out_type=jax.ShapeDtypeStruct((NC, N1), jnp.float32),
    mesh=_mesh,
    scratch_types=[
        pltpu.VMEM((CHUNKS, CHUNK), jnp.int32),
        pltpu.VMEM((CHUNK,), jnp.float32),
        pltpu.VMEM((RPT,), jnp.float32),
        pltpu.VMEM_SHARED((N1,), jnp.float32),
    ],
)
def _deg_kernel(dst_hbm, out_hbm, idx_v, ones_v, zero_v, deg_sh):
    c = lax.axis_index("c")
    s = lax.axis_index("s")
    w = c * NS + s

    for i in range(CHUNK // L):
        ones_v[pl.ds(i * L, L)] = jnp.ones((L,), jnp.float32)
    for i in range(RPT // L):
        zero_v[pl.ds(i * L, L)] = jnp.zeros((L,), jnp.float32)

    pltpu.sync_copy(dst_hbm.at[w], idx_v)
    pltpu.sync_copy(zero_v, deg_sh.at[pl.ds(s * RPT, RPT)])
    plsc.subcore_barrier()

    def body(j, carry):
        pltpu.sync_copy(ones_v, deg_sh.at[idx_v.at[j]], add=True)
        return carry

    lax.fori_loop(0, CHUNKS, body, 0)
    plsc.subcore_barrier()
    pltpu.sync_copy(deg_sh.at[pl.ds(s * RPT, RPT)],
                    out_hbm.at[c, pl.ds(s * RPT, RPT)])


@functools.partial(
    pl.kernel,
    out_type=jax.ShapeDtypeStruct((NC, N1, D), jnp.float32),
    mesh=_mesh,
    scratch_types=[
        pltpu.VMEM((CHUNKS, CHUNK), jnp.int32),
        pltpu.VMEM((BLK, CHUNK), jnp.int32),
        pltpu.VMEM((2, CHUNK, D), jnp.float32),
        pltpu.VMEM_SHARED((N1, D), jnp.float32),
        pltpu.SemaphoreType.DMA,
        pltpu.SemaphoreType.DMA,
    ],
)
def _agg_kernel(y_hbm, src_hbm, dst_hbm, out_hbm, src_v, dst_v, buf_v,
                acc_sh, gsem0, gsem1):
    c = lax.axis_index("c")
    s = lax.axis_index("s")
    w = c * NS + s
    base = s * RPT

    pltpu.sync_copy(src_hbm.at[w], src_v)
    pltpu.sync_copy(y_hbm.at[pl.ds(base, RPT)], acc_sh.at[pl.ds(base, RPT)])
    plsc.subcore_barrier()

    def gather(j, b, sem):
        return pltpu.make_async_copy(y_hbm.at[src_v.at[j]], buf_v.at[b], sem)

    def scatter(jb, b):
        pltpu.sync_copy(buf_v.at[b], acc_sh.at[dst_v.at[jb]], add=True)

    gather(0, 0, gsem0).start()
    gather(1, 1, gsem1).start()

    last_blk = CHUNKS // BLK - 1
    for blk in range(CHUNKS // BLK):
        pltpu.sync_copy(dst_hbm.at[w, pl.ds(blk * BLK, BLK)], dst_v)

        def pair(jj, carry, blk=blk):
            j0 = blk * BLK + 2 * jj
            j1 = j0 + 1
            gather(j0, 0, gsem0).wait()
            scatter(2 * jj, 0)
            if blk < last_blk:
                gather(j0 + 2, 0, gsem0).start()
            else:
                @pl.when(jj < BLK // 2 - 1)
                def _():
                    gather(j0 + 2, 0, gsem0).start()
            gather(j1, 1, gsem1).wait()
            scatter(2 * jj + 1, 1)
            if blk < last_blk:
                gather(j1 + 2, 1, gsem1).start()
            else:
                @pl.when(jj < BLK // 2 - 1)
                def _():
                    gather(j1 + 2, 1, gsem1).start()
            return carry

        lax.fori_loop(0, BLK // 2, pair, 0)

    plsc.subcore_barrier()
    pltpu.sync_copy(acc_sh.at[pl.ds(base, RPT)],
                    out_hbm.at[c, pl.ds(base, RPT)])


def _prescale_body(x_ref, w_ref, dp_ref, y_ref):
    xw = jnp.dot(x_ref[...], w_ref[...], preferred_element_type=jnp.float32)
    deg = dp_ref[0] + dp_ref[1] + 1.0
    y_ref[...] = xw * lax.rsqrt(deg)


def _epilogue_body(a_ref, y_ref, dp_ref, x_ref, b_ref, bw_ref, bb_ref,
                   bm_ref, bv_ref, o_ref):
    deg = dp_ref[0] + dp_ref[1] + 1.0
    dinv = lax.rsqrt(deg)
    agg = (a_ref[0] + a_ref[1] - y_ref[...]) * dinv + b_ref[...]
    inv_std = lax.rsqrt(bv_ref[...] + 1e-5)
    h = (agg - bm_ref[...]) * inv_std * bw_ref[...] + bb_ref[...]
    o_ref[...] = jnp.maximum(h, 0.0) + x_ref[...]


BR = 1024
_GRID = N1 // BR


def kernel(x, edge_index, W, b, bn_weight, bn_bias, bn_mean, bn_var):
    pad_e = E_PAD - E
    pad_idx = N + jnp.arange(pad_e, dtype=jnp.int32) % (N1 - N)
    src_flat = jnp.concatenate([edge_index[0], pad_idx])
    dst_flat = jnp.concatenate([edge_index[1], pad_idx])
    dst_t = dst_flat.reshape(NW, CHUNKS, CHUNK)
    src_t = src_flat.reshape(NW, CHUNKS, CHUNK)
    x_pad = jnp.pad(x, ((0, N1 - N), (0, 0)))

    deg_parts = _deg_kernel(dst_t)
    dp3 = deg_parts.reshape(NC, N1, 1)

    y = pl.pallas_call(
        _prescale_body,
        grid=(_GRID,),
        in_specs=[
            pl.BlockSpec((BR, D), lambda i: (i, 0)),
            pl.BlockSpec((D, D), lambda i: (0, 0)),
            pl.BlockSpec((NC, BR, 1), lambda i: (0, i, 0)),
        ],
        out_specs=pl.BlockSpec((BR, D), lambda i: (i, 0)),
        out_shape=jax.ShapeDtypeStruct((N1, D), jnp.float32),
    )(x_pad, W, dp3)

    agg_parts = _agg_kernel(y, src_t, dst_t)

    vec = lambda a: a.reshape(1, D)
    h = pl.pallas_call(
        _epilogue_body,
        grid=(_GRID,),
        in_specs=[
            pl.BlockSpec((NC, BR, D), lambda i: (0, i, 0)),
            pl.BlockSpec((BR, D), lambda i: (i, 0)),
            pl.BlockSpec((NC, BR, 1), lambda i: (0, i, 0)),
            pl.BlockSpec((BR, D), lambda i: (i, 0)),
        ] + [pl.BlockSpec((1, D), lambda i: (0, 0))] * 5,
        out_specs=pl.BlockSpec((BR, D), lambda i: (i, 0)),
        out_shape=jax.ShapeDtypeStruct((N1, D), jnp.float32),
    )(agg_parts, y, dp3, x_pad, vec(b), vec(bn_weight), vec(bn_bias),
      vec(bn_mean), vec(bn_var))

    return h[:N]
